# baseline (speedup 1.0000x reference)
import functools

import jax
import jax.numpy as jnp
from jax import lax
from jax.experimental import pallas as pl
from jax.experimental.pallas import tpu as pltpu
from jax.experimental.pallas import tpu_sc as plsc

N = 10000
E = 320000
D = 128
DH = 2 * D
L = 16
NC = 2
NS = 16
NW = NC * NS
K = 32
G = K // L
EPW = 10240
E_PAD = EPW * NW
NCHUNK = EPW // K
NCHUNKP = NCHUNK + 2
N_PAD = 10240
RPT = N_PAD // NS
DR = N_PAD // D
ZR = 16


def _prep_body(x_ref, sv_ref, w_ref, b_ref, hs_ref, sn_ref):
    h = lax.dot_general(x_ref[...], w_ref[...], (((1,), (1,)), ((), ())),
                        preferred_element_type=jnp.float32) + b_ref[...]
    s = sv_ref[...]
    nrm = jnp.maximum(jnp.sqrt(jnp.sum(s * s, axis=1, keepdims=True)), 1e-8)
    sn = s / nrm
    hs_ref[:, :D] = sn
    hs_ref[:, D:] = h
    sn_ref[...] = sn


def _prep(x, sv, w, b):
    blk = 1000
    return pl.pallas_call(
        _prep_body,
        grid=(N // blk,),
        in_specs=[
            pl.BlockSpec((blk, D), lambda i: (i, 0)),
            pl.BlockSpec((blk, D), lambda i: (i, 0)),
            pl.BlockSpec((D, D), lambda i: (0, 0)),
            pl.BlockSpec((1, D), lambda i: (0, 0)),
        ],
        out_specs=[
            pl.BlockSpec((blk, DH), lambda i: (i, 0)),
            pl.BlockSpec((blk, D), lambda i: (i, 0)),
        ],
        out_shape=[
            jax.ShapeDtypeStruct((N, DH), jnp.float32),
            jax.ShapeDtypeStruct((N, D), jnp.float32),
        ],
    )(x, sv, w, b)


def _sc_body(eix_hbm, sn_hbm, hs_hbm, acc_out, den_out,
             si, dsc, hs, bv, mb, dl, pidx, zb,
             acc_sh, gsem0, gsem1, ssem0, ssem1, isem0, isem1):
    c = lax.axis_index("c")
    s = lax.axis_index("s")
    wid = c * NS + s
    rbase = s * RPT
    zero = jnp.zeros((L,), jnp.float32)
    gsem = (gsem0, gsem1)
    ssem = (ssem0, ssem1)
    isem = (isem0, isem1)

    def zrow(i, _):
        for j in range(D // L):
            zb[i, pl.ds(j * L, L)] = zero
        return 0
    lax.fori_loop(0, ZR, zrow, 0)

    def zden(i, _):
        for j in range(D // L):
            dl[i, pl.ds(j * L, L)] = zero
        return 0
    lax.fori_loop(0, DR, zden, 0)

    for j in range(ZR // L):
        pidx[0, pl.ds(j * L, L)] = lax.iota(jnp.int32, L) + j * L

    for off in range(0, RPT, ZR):
        pltpu.sync_copy(zb, acc_sh.at[pl.ds(rbase + off, ZR)])
    plsc.subcore_barrier()

    def issue_idx(ci, slot):
        eb = (wid * NCHUNKP + ci) * (2 * K)
        pltpu.async_copy(eix_hbm.at[pl.ds(eb, 2 * K)], si.at[slot], isem[slot])

    def wait_idx(slot):
        pltpu.make_async_copy(
            eix_hbm.at[pl.ds(0, 2 * K)], si.at[slot], isem[slot]).wait()

    def issue_gathers(slot):
        pltpu.async_copy(hs_hbm.at[si.at[slot, pl.ds(0, K)]], hs.at[slot],
                         gsem[slot])
        pltpu.async_copy(sn_hbm.at[si.at[slot, pl.ds(K, K)]], bv.at[slot],
                         gsem[slot])

    def wait_gathers(slot):
        pltpu.make_async_copy(
            hs_hbm.at[si.at[slot, pl.ds(0, K)]], hs.at[slot], gsem[slot]).wait()
        pltpu.make_async_copy(
            sn_hbm.at[si.at[slot, pl.ds(K, K)]], bv.at[slot], gsem[slot]).wait()

    def issue_scatter(slot):
        pltpu.async_copy(mb.at[slot], acc_sh.at[dsc.at[slot]], ssem[slot],
                         add=True)

    def wait_scatter(slot):
        pltpu.make_async_copy(
            mb.at[slot], acc_sh.at[dsc.at[slot]], ssem[slot]).wait()

    issue_idx(0, 0)
    issue_idx(1, 1)
    pltpu.async_copy(zb, acc_sh.at[pidx.at[0]], ssem0, add=True)
    pltpu.async_copy(zb, acc_sh.at[pidx.at[0]], ssem0, add=True)
    pltpu.async_copy(zb, acc_sh.at[pidx.at[0]], ssem1, add=True)
    pltpu.async_copy(zb, acc_sh.at[pidx.at[0]], ssem1, add=True)
    wait_idx(0)
    issue_gathers(0)

    lanes = lax.iota(jnp.int32, L)

    def chunk(ci, slot):
        o = 1 - slot
        wait_gathers(slot)
        wait_scatter(o)
        wait_idx(o)
        issue_gathers(o)

        def group(g, _):
            dst16 = si[slot, pl.ds(K + g * L, L)]
            dsc[slot, pl.ds(g * L, L)] = dst16
            for half in range(2):
                ds = []
                for i in range(L // 2):
                    k = g * L + half * (L // 2) + i
                    m = [hs[slot, k, pl.ds(j * L, L)]
                         * bv[slot, k, pl.ds(j * L, L)] for j in range(D // L)]
                    ds.append(((m[0] + m[1]) + (m[2] + m[3]))
                              + ((m[4] + m[5]) + (m[6] + m[7])))
                z = jnp.zeros((L,), jnp.float32)
                for i in range(L // 2):
                    z = jnp.where(lanes == i, jnp.sum(ds[i]), z)
                pv = jnp.exp(z)
                for i in range(L // 2):
                    k = g * L + half * (L // 2) + i
                    p = jnp.full((L,), pv[i], jnp.float32)
                    for j in range(D // L):
                        mb[slot, k, pl.ds(j * L, L)] = (
                            hs[slot, k, pl.ds(D + j * L, L)] * p)
                    dk = dst16[half * (L // 2) + i]
                    plsc.addupdate_scatter(
                        dl, [jnp.full((L,), dk >> 7), jnp.full((L,), dk & 127)],
                        pv, mask=lanes == i)
            return 0
        lax.fori_loop(0, G, group, 0)

        issue_idx(ci + 2, slot)
        issue_scatter(slot)

    def outer(it, _):
        chunk(2 * it, 0)
        chunk(2 * it + 1, 1)
        return 0
    lax.fori_loop(0, NCHUNK // 2, outer, 0)

    wait_gathers(0)
    wait_idx(1)
    wait_scatter(0)
    wait_scatter(1)

    plsc.subcore_barrier()
    for off in range(0, RPT, ZR):
        pltpu.sync_copy(acc_sh.at[pl.ds(rbase + off, ZR)], zb)
        pltpu.sync_copy(zb, acc_out.at[c, pl.ds(rbase + off, ZR)])
    pltpu.sync_copy(dl, den_out.at[c, s])


_sc_pass = functools.partial(
    pl.kernel,
    out_type=[
        jax.ShapeDtypeStruct((NC, N_PAD, D), jnp.float32),
        jax.ShapeDtypeStruct((NC, NS, DR, D), jnp.float32),
    ],
    mesh=plsc.VectorSubcoreMesh(core_axis_name="c", subcore_axis_name="s"),
    compiler_params=pltpu.CompilerParams(needs_layout_passes=False),
    scratch_types=[
        pltpu.VMEM((2, 2 * K), jnp.int32),
        pltpu.VMEM((2, K), jnp.int32),
        pltpu.VMEM((2, K, DH), jnp.float32),
        pltpu.VMEM((2, K, D), jnp.float32),
        pltpu.VMEM((2, K, D), jnp.float32),
        pltpu.VMEM((DR, D), jnp.float32),
        pltpu.VMEM((1, ZR), jnp.int32),
        pltpu.VMEM((ZR, D), jnp.float32),
        pltpu.VMEM_SHARED((N_PAD, D), jnp.float32),
        pltpu.SemaphoreType.DMA,
        pltpu.SemaphoreType.DMA,
        pltpu.SemaphoreType.DMA,
        pltpu.SemaphoreType.DMA,
        pltpu.SemaphoreType.DMA,
        pltpu.SemaphoreType.DMA,
    ],
)(_sc_body)


def _combine_body(acc_ref, den_ref, out_ref):
    a = acc_ref[0] + acc_ref[1]
    ds = den_ref[0, 0]
    for cc in range(NC):
        for ss in range(NS):
            if cc or ss:
                ds = ds + den_ref[cc, ss]
    row = lax.broadcasted_iota(jnp.int32, (D, D), 0)
    col = lax.broadcasted_iota(jnp.int32, (D, D), 1)
    eye = jnp.where(row == col, 1.0, 0.0)
    dt = lax.dot_general(eye, ds, (((1,), (1,)), ((), ())),
                         preferred_element_type=jnp.float32)
    for gi in range(ds.shape[0]):
        ablk = a[gi * D:(gi + 1) * D]
        dcol = dt[:, gi:gi + 1]
        out_ref[gi * D:(gi + 1) * D, :] = jnp.where(dcol > 0, ablk / dcol, 0.0)


def _combine(acc, den):
    blk = 1024
    return pl.pallas_call(
        _combine_body,
        grid=(N_PAD // blk,),
        in_specs=[
            pl.BlockSpec((NC, blk, D), lambda i: (0, i, 0)),
            pl.BlockSpec((NC, NS, blk // D, D), lambda i: (0, 0, i, 0)),
        ],
        out_specs=pl.BlockSpec((blk, D), lambda i: (i, 0)),
        out_shape=jax.ShapeDtypeStruct((N_PAD, D), jnp.float32),
    )(acc, den)


def kernel(x, edge_index, semantic_vec, W_src, b_src, W_dst, b_dst):
    del W_dst, b_dst
    src = edge_index[0]
    dst = edge_index[1]
    pad = E_PAD - E
    src_r = jnp.concatenate([src, jnp.zeros((pad,), src.dtype)]).reshape(
        NW, NCHUNK, K)
    dst_r = jnp.concatenate([dst, jnp.full((pad,), N, dst.dtype)]).reshape(
        NW, NCHUNK, K)
    src_x = jnp.concatenate([src_r, jnp.zeros((NW, 2, K), src.dtype)], axis=1)
    dst_x = jnp.concatenate([dst_r, jnp.full((NW, 2, K), N, dst.dtype)], axis=1)
    eix = jnp.stack([src_x, dst_x], axis=2).reshape(-1)
    hs, sn = _prep(x, semantic_vec, W_src, b_src.reshape(1, D))
    acc, den = _sc_pass(eix, sn, hs)
    return _combine(acc, den)[:N]


# speedup vs baseline: 1.4020x; 1.4020x over previous
import functools

import jax
import jax.numpy as jnp
from jax import lax
from jax.experimental import pallas as pl
from jax.experimental.pallas import tpu as pltpu
from jax.experimental.pallas import tpu_sc as plsc

N = 10000
E = 320000
D = 128
L = 16
NC = 2
NS = 16
NW = NC * NS
K = 32
G = K // L
EPW = 10240
E_PAD = EPW * NW
NCHUNK = EPW // K
N_PAD = 10240
RPT = N_PAD // NS
DR = N_PAD // D


def _prep_body(x_ref, sv_ref, w_ref, b_ref, h_ref, sn_ref):
    h_ref[...] = lax.dot_general(x_ref[...], w_ref[...], (((1,), (1,)), ((), ())),
                                 preferred_element_type=jnp.float32) + b_ref[...]
    s = sv_ref[...]
    nrm = jnp.maximum(jnp.sqrt(jnp.sum(s * s, axis=1, keepdims=True)), 1e-8)
    sn_ref[...] = (s / nrm).astype(jnp.bfloat16)


def _prep(x, sv, w, b):
    blk = 1000
    return pl.pallas_call(
        _prep_body,
        grid=(N // blk,),
        in_specs=[
            pl.BlockSpec((blk, D), lambda i: (i, 0)),
            pl.BlockSpec((blk, D), lambda i: (i, 0)),
            pl.BlockSpec((D, D), lambda i: (0, 0)),
            pl.BlockSpec((1, D), lambda i: (0, 0)),
        ],
        out_specs=[
            pl.BlockSpec((blk, D), lambda i: (i, 0)),
            pl.BlockSpec((blk, D), lambda i: (i, 0)),
        ],
        out_shape=[
            jax.ShapeDtypeStruct((N, D), jnp.float32),
            jax.ShapeDtypeStruct((N, D), jnp.bfloat16),
        ],
    )(x, sv, w, b)


def _sc_body(src_hbm, dst_hbm, sn_hbm, h_hbm, acc_out, den_out,
             si, di, dsc, av, bv, hv, dl, idb, pidx, zb,
             acc_sh, den_sh, gsem0, gsem1, ssem0, ssem1, isem0, isem1):
    c = lax.axis_index("c")
    s = lax.axis_index("s")
    wid = c * NS + s
    ebase = wid * EPW
    rbase = s * RPT
    zero = jnp.zeros((L,), jnp.float32)
    lane0 = lax.iota(jnp.int32, L) == 0
    gsem = (gsem0, gsem1)
    ssem = (ssem0, ssem1)
    isem = (isem0, isem1)

    def zrow(i, _):
        for j in range(D // L):
            zb[i, pl.ds(j * L, L)] = zero
        return 0
    lax.fori_loop(0, K, zrow, 0)

    def zden(i, _):
        for j in range(D // L):
            dl[i, pl.ds(j * L, L)] = zero
        return 0
    lax.fori_loop(0, DR, zden, 0)

    for j in range(DR // L):
        idb[0, pl.ds(j * L, L)] = lax.iota(jnp.int32, L) + j * L
    for j in range(K // L):
        pidx[0, pl.ds(j * L, L)] = lax.iota(jnp.int32, L) + j * L

    for off in range(0, RPT, K):
        pltpu.sync_copy(zb, acc_sh.at[pl.ds(rbase + off, K)])

    @pl.when(s == 0)
    def _():
        pltpu.sync_copy(dl, den_sh.at[pl.ds(0, DR)])
    plsc.subcore_barrier()

    def issue_idx(ci, slot):
        eb = ebase + ci * K
        pltpu.async_copy(src_hbm.at[pl.ds(eb, K)], si.at[slot], isem[slot])
        pltpu.async_copy(dst_hbm.at[pl.ds(eb, K)], di.at[slot], isem[slot])

    def wait_idx(slot):
        pltpu.make_async_copy(src_hbm.at[pl.ds(0, K)], si.at[slot], isem[slot]).wait()
        pltpu.make_async_copy(dst_hbm.at[pl.ds(0, K)], di.at[slot], isem[slot]).wait()

    def issue_gathers(slot):
        pltpu.async_copy(sn_hbm.at[si.at[slot]], av.at[slot], gsem[slot])
        pltpu.async_copy(sn_hbm.at[di.at[slot]], bv.at[slot], gsem[slot])
        pltpu.async_copy(h_hbm.at[si.at[slot]], hv.at[slot], gsem[slot])

    def wait_gathers(slot):
        pltpu.make_async_copy(sn_hbm.at[si.at[slot]], av.at[slot], gsem[slot]).wait()
        pltpu.make_async_copy(sn_hbm.at[di.at[slot]], bv.at[slot], gsem[slot]).wait()
        pltpu.make_async_copy(h_hbm.at[si.at[slot]], hv.at[slot], gsem[slot]).wait()

    def wait_scatter(slot):
        pltpu.make_async_copy(
            hv.at[slot], acc_sh.at[dsc.at[slot]], ssem[slot]).wait()

    issue_idx(0, 0)
    issue_idx(1, 1)
    pltpu.async_copy(zb, acc_sh.at[pidx.at[0]], ssem0, add=True)
    pltpu.async_copy(zb, acc_sh.at[pidx.at[0]], ssem1, add=True)
    wait_idx(0)
    issue_gathers(0)

    def chunk(ci, slot):
        o = 1 - slot
        wait_gathers(slot)
        wait_scatter(o)
        wait_idx(o)
        issue_gathers(o)

        lanes = lax.iota(jnp.int32, L)

        def group(g, _):
            dst16 = di[slot, pl.ds(g * L, L)]
            dsc[slot, pl.ds(g * L, L)] = dst16
            ds = []
            for i in range(L):
                k = g * L + i
                m = []
                for j in range(D // (2 * L)):
                    ae, ao = plsc.unpack(av[slot, k, pl.ds(j * 2 * L, 2 * L)],
                                         format=plsc.PackFormat.INTERLEAVED)
                    be, bo = plsc.unpack(bv[slot, k, pl.ds(j * 2 * L, 2 * L)],
                                         format=plsc.PackFormat.INTERLEAVED)
                    m.append(ae * be)
                    m.append(ao * bo)
                ds.append(((m[0] + m[1]) + (m[2] + m[3]))
                          + ((m[4] + m[5]) + (m[6] + m[7])))
            z = jnp.zeros((L,), jnp.float32)
            for i in range(L):
                z = jnp.where(lanes == i, jnp.sum(ds[i]), z)
            pv = jnp.exp(z)
            for i in range(L):
                k = g * L + i
                p = jnp.full((L,), pv[i], jnp.float32)
                for j in range(D // L):
                    hv[slot, k, pl.ds(j * L, L)] = hv[slot, k, pl.ds(j * L, L)] * p
                dk = dst16[i]
                plsc.addupdate_scatter(
                    dl, [jnp.full((L,), dk >> 7), jnp.full((L,), dk & 127)],
                    pv, mask=lanes == i)
            return 0
        lax.fori_loop(0, G, group, 0)

        issue_idx(ci + 2, slot)
        pltpu.async_copy(hv.at[slot], acc_sh.at[dsc.at[slot]], ssem[slot],
                         add=True)

    def outer(it, _):
        chunk(2 * it, 0)
        chunk(2 * it + 1, 1)
        return 0
    lax.fori_loop(0, NCHUNK // 2, outer, 0)

    wait_gathers(0)
    wait_idx(1)
    wait_scatter(0)
    wait_scatter(1)

    pltpu.sync_copy(dl, den_sh.at[idb.at[0]], add=True)
    plsc.subcore_barrier()
    for off in range(0, RPT, K):
        pltpu.sync_copy(acc_sh.at[pl.ds(rbase + off, K)], zb)
        pltpu.sync_copy(zb, acc_out.at[c, pl.ds(rbase + off, K)])

    @pl.when(s == 0)
    def _():
        pltpu.sync_copy(den_sh.at[pl.ds(0, DR)], dl)
        pltpu.sync_copy(dl, den_out.at[c])


_sc_pass = functools.partial(
    pl.kernel,
    out_type=[
        jax.ShapeDtypeStruct((NC, N_PAD, D), jnp.float32),
        jax.ShapeDtypeStruct((NC, DR, D), jnp.float32),
    ],
    mesh=plsc.VectorSubcoreMesh(core_axis_name="c", subcore_axis_name="s"),
    compiler_params=pltpu.CompilerParams(needs_layout_passes=False,
                                         use_tc_tiling_on_sc=False),
    scratch_types=[
        pltpu.VMEM((2, K), jnp.int32),
        pltpu.VMEM((2, K), jnp.int32),
        pltpu.VMEM((2, K), jnp.int32),
        pltpu.VMEM((2, K, D), jnp.bfloat16),
        pltpu.VMEM((2, K, D), jnp.bfloat16),
        pltpu.VMEM((2, K, D), jnp.float32),
        pltpu.VMEM((DR, D), jnp.float32),
        pltpu.VMEM((1, DR), jnp.int32),
        pltpu.VMEM((1, K), jnp.int32),
        pltpu.VMEM((K, D), jnp.float32),
        pltpu.VMEM_SHARED((N_PAD, D), jnp.float32),
        pltpu.VMEM_SHARED((DR, D), jnp.float32),
        pltpu.SemaphoreType.DMA,
        pltpu.SemaphoreType.DMA,
        pltpu.SemaphoreType.DMA,
        pltpu.SemaphoreType.DMA,
        pltpu.SemaphoreType.DMA,
        pltpu.SemaphoreType.DMA,
    ],
)(_sc_body)


def _combine_body(acc_ref, den_ref, out_ref):
    a = acc_ref[0] + acc_ref[1]
    ds = den_ref[0] + den_ref[1]
    row = lax.broadcasted_iota(jnp.int32, (D, D), 0)
    col = lax.broadcasted_iota(jnp.int32, (D, D), 1)
    eye = jnp.where(row == col, 1.0, 0.0)
    dt = lax.dot_general(eye, ds, (((1,), (1,)), ((), ())),
                         preferred_element_type=jnp.float32)
    for gi in range(ds.shape[0]):
        ablk = a[gi * D:(gi + 1) * D]
        dcol = dt[:, gi:gi + 1]
        out_ref[gi * D:(gi + 1) * D, :] = jnp.where(dcol > 0, ablk / dcol, 0.0)


def _combine(acc, den):
    blk = 1024
    return pl.pallas_call(
        _combine_body,
        grid=(N_PAD // blk,),
        in_specs=[
            pl.BlockSpec((NC, blk, D), lambda i: (0, i, 0)),
            pl.BlockSpec((NC, blk // D, D), lambda i: (0, i, 0)),
        ],
        out_specs=pl.BlockSpec((blk, D), lambda i: (i, 0)),
        out_shape=jax.ShapeDtypeStruct((N_PAD, D), jnp.float32),
    )(acc, den)


def kernel(x, edge_index, semantic_vec, W_src, b_src, W_dst, b_dst):
    del W_dst, b_dst
    src = edge_index[0]
    dst = edge_index[1]
    pad = E_PAD + 2 * K - E
    src_p = jnp.concatenate([src, jnp.zeros((pad,), src.dtype)])
    dst_p = jnp.concatenate([dst, jnp.full((pad,), N, dst.dtype)])
    h, sn = _prep(x, semantic_vec, W_src, b_src.reshape(1, D))
    acc, den = _sc_pass(src_p, dst_p, sn, h)
    return _combine(acc, den)[:N]


# speedup vs baseline: 1.6750x; 1.1948x over previous
import functools

import jax
import jax.numpy as jnp
from jax import lax
from jax.experimental import pallas as pl
from jax.experimental.pallas import tpu as pltpu
from jax.experimental.pallas import tpu_sc as plsc

N = 10000
E = 320000
D = 128
L = 16
NC = 2
NS = 16
NW = NC * NS
K = 32
G = K // L
EPW = 10240
E_PAD = EPW * NW
NCHUNK = EPW // K
N_PAD = 10240
RPT = N_PAD // NS
DR = N_PAD // D


def _prep_body(x_ref, sv_ref, w_ref, b_ref, h_ref, sn_ref):
    h_ref[...] = (lax.dot_general(x_ref[...], w_ref[...], (((1,), (1,)), ((), ())),
                                  preferred_element_type=jnp.float32)
                  + b_ref[...]).astype(jnp.bfloat16)
    s = sv_ref[...]
    nrm = jnp.maximum(jnp.sqrt(jnp.sum(s * s, axis=1, keepdims=True)), 1e-8)
    sn_ref[...] = (s / nrm).astype(jnp.bfloat16)


def _prep(x, sv, w, b):
    blk = 1000
    return pl.pallas_call(
        _prep_body,
        grid=(N // blk,),
        in_specs=[
            pl.BlockSpec((blk, D), lambda i: (i, 0)),
            pl.BlockSpec((blk, D), lambda i: (i, 0)),
            pl.BlockSpec((D, D), lambda i: (0, 0)),
            pl.BlockSpec((1, D), lambda i: (0, 0)),
        ],
        out_specs=[
            pl.BlockSpec((blk, D), lambda i: (i, 0)),
            pl.BlockSpec((blk, D), lambda i: (i, 0)),
        ],
        out_shape=[
            jax.ShapeDtypeStruct((N, D), jnp.bfloat16),
            jax.ShapeDtypeStruct((N, D), jnp.bfloat16),
        ],
    )(x, sv, w, b)


def _sc_body(src_hbm, dst_hbm, sn_hbm, h_hbm, acc_out, den_out,
             si, di, dsc, av, bv, hv, mb, dl, idb, pidx, zb,
             acc_sh, den_sh, gsem0, gsem1, ssem0, ssem1, isem0, isem1):
    c = lax.axis_index("c")
    s = lax.axis_index("s")
    wid = c * NS + s
    ebase = wid * EPW
    rbase = s * RPT
    zero = jnp.zeros((L,), jnp.float32)
    lane0 = lax.iota(jnp.int32, L) == 0
    gsem = (gsem0, gsem1)
    ssem = (ssem0, ssem1)
    isem = (isem0, isem1)

    def zrow(i, _):
        for j in range(D // L):
            zb[i, pl.ds(j * L, L)] = zero
        return 0
    lax.fori_loop(0, K, zrow, 0)

    def zden(i, _):
        for j in range(D // L):
            dl[i, pl.ds(j * L, L)] = zero
        return 0
    lax.fori_loop(0, DR, zden, 0)

    for j in range(DR // L):
        idb[0, pl.ds(j * L, L)] = lax.iota(jnp.int32, L) + j * L
    for j in range(K // L):
        pidx[0, pl.ds(j * L, L)] = lax.iota(jnp.int32, L) + j * L

    for off in range(0, RPT, K):
        pltpu.sync_copy(zb, acc_sh.at[pl.ds(rbase + off, K)])

    @pl.when(s == 0)
    def _():
        pltpu.sync_copy(dl, den_sh.at[pl.ds(0, DR)])
    plsc.subcore_barrier()

    def issue_idx(ci, slot):
        eb = ebase + ci * K
        pltpu.async_copy(src_hbm.at[pl.ds(eb, K)], si.at[slot], isem[slot])
        pltpu.async_copy(dst_hbm.at[pl.ds(eb, K)], di.at[slot], isem[slot])

    def wait_idx(slot):
        pltpu.make_async_copy(src_hbm.at[pl.ds(0, K)], si.at[slot], isem[slot]).wait()
        pltpu.make_async_copy(dst_hbm.at[pl.ds(0, K)], di.at[slot], isem[slot]).wait()

    def issue_gathers(slot):
        pltpu.async_copy(sn_hbm.at[si.at[slot]], av.at[slot], gsem[slot])
        pltpu.async_copy(sn_hbm.at[di.at[slot]], bv.at[slot], gsem[slot])
        pltpu.async_copy(h_hbm.at[si.at[slot]], hv.at[slot], gsem[slot])

    def wait_gathers(slot):
        pltpu.make_async_copy(sn_hbm.at[si.at[slot]], av.at[slot], gsem[slot]).wait()
        pltpu.make_async_copy(sn_hbm.at[di.at[slot]], bv.at[slot], gsem[slot]).wait()
        pltpu.make_async_copy(h_hbm.at[si.at[slot]], hv.at[slot], gsem[slot]).wait()

    def wait_scatter(slot):
        pltpu.make_async_copy(
            mb.at[slot], acc_sh.at[dsc.at[slot]], ssem[slot]).wait()

    issue_idx(0, 0)
    issue_idx(1, 1)
    pltpu.async_copy(zb, acc_sh.at[pidx.at[0]], ssem0, add=True)
    pltpu.async_copy(zb, acc_sh.at[pidx.at[0]], ssem1, add=True)
    wait_idx(0)
    issue_gathers(0)

    def chunk(ci, slot):
        o = 1 - slot
        wait_gathers(slot)
        wait_scatter(o)
        wait_idx(o)
        issue_gathers(o)

        lanes = lax.iota(jnp.int32, L)

        def group(g, _):
            dst16 = di[slot, pl.ds(g * L, L)]
            dsc[slot, pl.ds(g * L, L)] = dst16
            ds = []
            for i in range(L):
                k = g * L + i
                m = []
                for j in range(D // (2 * L)):
                    ae, ao = plsc.unpack(av[slot, k, pl.ds(j * 2 * L, 2 * L)],
                                         format=plsc.PackFormat.INTERLEAVED)
                    be, bo = plsc.unpack(bv[slot, k, pl.ds(j * 2 * L, 2 * L)],
                                         format=plsc.PackFormat.INTERLEAVED)
                    m.append(ae * be)
                    m.append(ao * bo)
                ds.append(((m[0] + m[1]) + (m[2] + m[3]))
                          + ((m[4] + m[5]) + (m[6] + m[7])))
            z = jnp.zeros((L,), jnp.float32)
            for i in range(L):
                z = jnp.where(lanes == i, jnp.sum(ds[i]), z)
            pv = jnp.exp(z)
            for i in range(L):
                k = g * L + i
                p = jnp.full((L,), pv[i], jnp.float32)
                for j in range(D // (2 * L)):
                    he, ho = plsc.unpack(hv[slot, k, pl.ds(j * 2 * L, 2 * L)],
                                         format=plsc.PackFormat.INTERLEAVED)
                    mb[slot, k, pl.ds(j * L, L)] = he * p
                    mb[slot, k, pl.ds(D // 2 + j * L, L)] = ho * p
                dk = dst16[i]
                plsc.addupdate_scatter(
                    dl, [jnp.full((L,), dk >> 7), jnp.full((L,), dk & 127)],
                    pv, mask=lanes == i)
            return 0
        lax.fori_loop(0, G, group, 0)

        issue_idx(ci + 2, slot)
        pltpu.async_copy(mb.at[slot], acc_sh.at[dsc.at[slot]], ssem[slot],
                         add=True)

    def outer(it, _):
        chunk(2 * it, 0)
        chunk(2 * it + 1, 1)
        return 0
    lax.fori_loop(0, NCHUNK // 2, outer, 0)

    wait_gathers(0)
    wait_idx(1)
    wait_scatter(0)
    wait_scatter(1)

    pltpu.sync_copy(dl, den_sh.at[idb.at[0]], add=True)
    plsc.subcore_barrier()
    for off in range(0, RPT, K):
        pltpu.sync_copy(acc_sh.at[pl.ds(rbase + off, K)], zb)
        pltpu.sync_copy(zb, acc_out.at[c, pl.ds(rbase + off, K)])

    @pl.when(s == 0)
    def _():
        pltpu.sync_copy(den_sh.at[pl.ds(0, DR)], dl)
        pltpu.sync_copy(dl, den_out.at[c])


_sc_pass = functools.partial(
    pl.kernel,
    out_type=[
        jax.ShapeDtypeStruct((NC, N_PAD, D), jnp.float32),
        jax.ShapeDtypeStruct((NC, DR, D), jnp.float32),
    ],
    mesh=plsc.VectorSubcoreMesh(core_axis_name="c", subcore_axis_name="s"),
    compiler_params=pltpu.CompilerParams(needs_layout_passes=False,
                                         use_tc_tiling_on_sc=False),
    scratch_types=[
        pltpu.VMEM((2, K), jnp.int32),
        pltpu.VMEM((2, K), jnp.int32),
        pltpu.VMEM((2, K), jnp.int32),
        pltpu.VMEM((2, K, D), jnp.bfloat16),
        pltpu.VMEM((2, K, D), jnp.bfloat16),
        pltpu.VMEM((2, K, D), jnp.bfloat16),
        pltpu.VMEM((2, K, D), jnp.float32),
        pltpu.VMEM((DR, D), jnp.float32),
        pltpu.VMEM((1, DR), jnp.int32),
        pltpu.VMEM((1, K), jnp.int32),
        pltpu.VMEM((K, D), jnp.float32),
        pltpu.VMEM_SHARED((N_PAD, D), jnp.float32),
        pltpu.VMEM_SHARED((DR, D), jnp.float32),
        pltpu.SemaphoreType.DMA,
        pltpu.SemaphoreType.DMA,
        pltpu.SemaphoreType.DMA,
        pltpu.SemaphoreType.DMA,
        pltpu.SemaphoreType.DMA,
        pltpu.SemaphoreType.DMA,
    ],
)(_sc_body)


def _combine_body(acc_ref, den_ref, out_ref):
    a = acc_ref[0] + acc_ref[1]
    ds = den_ref[0] + den_ref[1]
    row = lax.broadcasted_iota(jnp.int32, (D, D), 0)
    col = lax.broadcasted_iota(jnp.int32, (D, D), 1)
    eye = jnp.where(row == col, 1.0, 0.0)
    dt = lax.dot_general(eye, ds, (((1,), (1,)), ((), ())),
                         preferred_element_type=jnp.float32)
    orig = jnp.where(row < D // 2, 2 * row, 2 * (row - D // 2) + 1)
    perm = jnp.where(col == orig, 1.0, 0.0)
    for gi in range(ds.shape[0]):
        ablk = a[gi * D:(gi + 1) * D]
        dcol = dt[:, gi:gi + 1]
        r = jnp.where(dcol > 0, ablk / dcol, 0.0)
        out_ref[gi * D:(gi + 1) * D, :] = lax.dot_general(
            r, perm, (((1,), (0,)), ((), ())),
            preferred_element_type=jnp.float32)


def _combine(acc, den):
    blk = 1024
    return pl.pallas_call(
        _combine_body,
        grid=(N_PAD // blk,),
        in_specs=[
            pl.BlockSpec((NC, blk, D), lambda i: (0, i, 0)),
            pl.BlockSpec((NC, blk // D, D), lambda i: (0, i, 0)),
        ],
        out_specs=pl.BlockSpec((blk, D), lambda i: (i, 0)),
        out_shape=jax.ShapeDtypeStruct((N_PAD, D), jnp.float32),
    )(acc, den)


def kernel(x, edge_index, semantic_vec, W_src, b_src, W_dst, b_dst):
    del W_dst, b_dst
    src = edge_index[0]
    dst = edge_index[1]
    pad = E_PAD + 2 * K - E
    src_p = jnp.concatenate([src, jnp.zeros((pad,), src.dtype)])
    dst_p = jnp.concatenate([dst, jnp.full((pad,), N, dst.dtype)])
    h, sn = _prep(x, semantic_vec, W_src, b_src.reshape(1, D))
    acc, den = _sc_pass(src_p, dst_p, sn, h)
    return _combine(acc, den)[:N]
